# baseline (speedup 1.0000x reference)
import functools

import jax
import jax.numpy as jnp
from jax.experimental import pallas as pl
from jax.experimental.pallas import tpu as pltpu

B = 2
L = 2048
D = 256
PRED = 720
PATCH = 64
NPATCH = L // PATCH
E = 8
FF = 2048
EPS = 1e-5
ALPHA = 1.0
R = B * D

_LENS = [256, 512, 1024, 2048]


def _dotb(a, b):
    return jnp.dot(a.astype(jnp.bfloat16), b.astype(jnp.bfloat16),
                   preferred_element_type=jnp.float32)


def _gelu(x):
    return jax.nn.gelu(x, approximate=True)


def _bn_rows(x, g_col, be_col, length):
    s1 = jnp.sum(x, axis=1, keepdims=True)
    s2 = jnp.sum(x * x, axis=1, keepdims=True)
    ps1 = s1[:D] + s1[D:]
    ps2 = s2[:D] + s2[D:]
    n = float(B) * length
    m = ps1 / n
    v = ps2 / n - m * m
    scale = jax.lax.rsqrt(v + EPS) * g_col
    shift = be_col - m * scale
    scale2 = jnp.concatenate([scale, scale], axis=0)
    shift2 = jnp.concatenate([shift, shift], axis=0)
    return x * scale2 + shift2


def _pool_mat(n_in, n_out):
    k = n_in // n_out
    ri = jax.lax.broadcasted_iota(jnp.int32, (n_in, n_out), 0)
    ci = jax.lax.broadcasted_iota(jnp.int32, (n_in, n_out), 1)
    return jnp.where(ri // k == ci, 1.0 / k, 0.0).astype(jnp.float32)


def _pool_exact(x, n_out):
    pm = _pool_mat(L, n_out)
    hi = x.astype(jnp.bfloat16).astype(jnp.float32)
    lo = x - hi
    return _dotb(hi, pm) + _dotb(lo, pm)


def _mdm_layer(out, sample, w1, b1, w2, b2, g, be, length):
    h = _gelu(_dotb(out, w1) + b1)
    out = _dotb(h, w2) + b2
    out = out + sample
    return _bn_rows(out, g, be, length)


def _backbone_kernel(x_ref, rw_ref, rb_ref,
                     w10, b10, w20, b20, g0, be0,
                     w11, b11, w21, b21, g1, be1,
                     w12, b12, w22, b22, g2, be2,
                     gw_ref, gb_ref,
                     xn_ref, gates_ref, loss_ref, mean_ref, std_ref):
    X = x_ref[...]
    m = jnp.mean(X, axis=1, keepdims=True)
    ex2 = jnp.mean(X * X, axis=1, keepdims=True)
    std = jnp.sqrt(ex2 - m * m + EPS)
    mean_ref[...] = m
    std_ref[...] = std
    Xn = (X - m) / std * rw_ref[...] + rb_ref[...]
    xn_ref[...] = Xn

    s8 = _pool_exact(Xn, 256)
    s4 = _pool_exact(Xn, 512)
    s2 = _pool_exact(Xn, 1024)
    out = _mdm_layer(s8, s4, w10[...], b10[...], w20[...], b20[...],
                     g0[...], be0[...], float(_LENS[1]))
    out = _mdm_layer(out, s2, w11[...], b11[...], w21[...], b21[...],
                     g1[...], be1[...], float(_LENS[2]))
    te = _mdm_layer(out, Xn, w12[...], b12[...], w22[...], b22[...],
                    g2[...], be2[...], float(_LENS[3]))

    logits = _dotb(te, gw_ref[...]) + gb_ref[...]
    iota = jax.lax.broadcasted_iota(jnp.int32, (R, E), 1)
    m1 = jnp.max(logits, axis=1, keepdims=True)
    i1 = jnp.min(jnp.where(logits == m1, iota, E), axis=1, keepdims=True)
    masked = jnp.where(iota == i1, -jnp.inf, logits)
    m2 = jnp.max(masked, axis=1, keepdims=True)
    i2 = jnp.min(jnp.where(masked == m2, iota, E), axis=1, keepdims=True)
    e2 = jnp.exp(m2 - m1)
    g0v = 1.0 / (1.0 + e2)
    g1v = 1.0 - g0v
    gates = jnp.where(iota == i1, g0v, 0.0) + jnp.where(iota == i2, g1v, 0.0)
    gates_ref[...] = gates
    imp = jnp.sum(gates, axis=0, keepdims=True)
    im = jnp.sum(imp, axis=1, keepdims=True) / E
    iv = jnp.sum(imp * imp, axis=1, keepdims=True) / E - im * im
    loss_ref[...] = iv / (im * im + 1e-10)


def _ddi_kernel(x_ref, w1, b1, w2, b2, wc1t, bc1, wc2t, bc2, g, be, o_ref):
    X = x_ref[...]
    xcat = jnp.concatenate(
        [X[:, j * PATCH:(j + 1) * PATCH] for j in range(NPATCH)], axis=0)
    h = _gelu(_dotb(xcat, w1[...]) + b1[...])
    u = _dotb(h, w2[...]) + b2[...]
    X = X + jnp.concatenate(
        [u[j * R:(j + 1) * R] for j in range(NPATCH)], axis=1)
    outs = []
    for bi in range(B):
        Xb = X[bi * D:(bi + 1) * D]
        Y = _gelu(_dotb(wc1t[...], Xb) + bc1[...])
        V = _dotb(wc2t[...], Y) + bc2[...]
        outs.append(Xb + ALPHA * V)
    X = jnp.concatenate(outs, axis=0)
    o_ref[...] = _bn_rows(X, g[...].T, be[...].T, float(L))


def _moe_kernel(tok_ref, gates_ref, ew1, eb1, ew2, eb2,
                mean_ref, std_ref, rw_ref, rb_ref, out_ref):
    e = pl.program_id(0)
    f = pl.program_id(1)
    nf = pl.num_programs(1)

    @pl.when(jnp.logical_and(e == 0, f == 0))
    def _init():
        out_ref[...] = jnp.zeros_like(out_ref)

    tok = tok_ref[...]
    h = _gelu(_dotb(tok, ew1[0]) + eb1[0])
    part = _dotb(h, ew2[0])
    ei = jax.lax.broadcasted_iota(jnp.int32, (R, E), 1)
    g = jnp.sum(jnp.where(ei == e, gates_ref[...], 0.0), axis=1, keepdims=True)

    @pl.when(f == 0)
    def _bias():
        out_ref[...] += g * eb2[0]

    out_ref[...] += g * part

    @pl.when(jnp.logical_and(e == E - 1, f == nf - 1))
    def _finish():
        o = out_ref[...]
        o = (o - rb_ref[...]) / rw_ref[...] * std_ref[...] + mean_ref[...]
        out_ref[...] = o


def kernel(x, params):
    p = params
    bf = jnp.bfloat16
    xt = jnp.transpose(x, (0, 2, 1)).reshape(R, L)
    rw = jnp.tile(p['revin_w'], B)[:, None]
    rb = jnp.tile(p['revin_b'], B)[:, None]

    vmem = functools.partial(pl.BlockSpec, memory_space=pltpu.VMEM)

    def layer_args(blk):
        return [blk['w1'].astype(bf), blk['b1'][None, :],
                blk['w2'].astype(bf), blk['b2'][None, :],
                blk['g'][:, None], blk['be'][:, None]]

    mdm = p['mdm']
    xn, gates, loss, mean, std = pl.pallas_call(
        _backbone_kernel,
        out_shape=[
            jax.ShapeDtypeStruct((R, L), jnp.float32),
            jax.ShapeDtypeStruct((R, E), jnp.float32),
            jax.ShapeDtypeStruct((1, 1), jnp.float32),
            jax.ShapeDtypeStruct((R, 1), jnp.float32),
            jax.ShapeDtypeStruct((R, 1), jnp.float32),
        ],
        compiler_params=pltpu.CompilerParams(
            vmem_limit_bytes=110 * 1024 * 1024),
    )(xt, rw, rb, *layer_args(mdm[0]), *layer_args(mdm[1]),
      *layer_args(mdm[2]), p['gate_w'].astype(bf), p['gate_b'][None, :])

    cur = xn
    for blk in p['ddi']:
        cur = pl.pallas_call(
            _ddi_kernel,
            out_shape=jax.ShapeDtypeStruct((R, L), jnp.float32),
            compiler_params=pltpu.CompilerParams(
                vmem_limit_bytes=110 * 1024 * 1024),
        )(cur, blk['w1'].astype(bf), blk['b1'][None, :],
          blk['w2'].astype(bf), blk['b2'][None, :],
          blk['wc1'].T.astype(bf), blk['bc1'][:, None],
          blk['wc2'].T.astype(bf), blk['bc2'][:, None],
          blk['g'][None, :], blk['be'][None, :])

    FBLK = 512
    nf = FF // FBLK
    out_tok = pl.pallas_call(
        _moe_kernel,
        grid=(E, nf),
        in_specs=[
            vmem((R, L), lambda e, f: (0, 0)),
            vmem((R, E), lambda e, f: (0, 0)),
            vmem((1, L, FBLK), lambda e, f: (e, 0, f)),
            vmem((1, 1, FBLK), lambda e, f: (e, 0, f)),
            vmem((1, FBLK, PRED), lambda e, f: (e, f, 0)),
            vmem((1, 1, PRED), lambda e, f: (e, 0, 0)),
            vmem((R, 1), lambda e, f: (0, 0)),
            vmem((R, 1), lambda e, f: (0, 0)),
            vmem((R, 1), lambda e, f: (0, 0)),
            vmem((R, 1), lambda e, f: (0, 0)),
        ],
        out_specs=vmem((R, PRED), lambda e, f: (0, 0)),
        out_shape=jax.ShapeDtypeStruct((R, PRED), jnp.float32),
        compiler_params=pltpu.CompilerParams(
            dimension_semantics=("arbitrary", "arbitrary"),
            vmem_limit_bytes=100 * 1024 * 1024),
    )(cur, gates, p['ew1'].astype(bf), p['eb1'][:, None, :],
      p['ew2'].astype(bf), p['eb2'][:, None, :], mean, std, rw, rb)

    out = jnp.transpose(out_tok.reshape(B, D, PRED), (0, 2, 1))
    return out, loss.reshape(())


# speedup vs baseline: 1.0396x; 1.0396x over previous
import functools

import jax
import jax.numpy as jnp
from jax.experimental import pallas as pl
from jax.experimental.pallas import tpu as pltpu

B = 2
L = 2048
D = 256
PRED = 720
PATCH = 64
NPATCH = L // PATCH
E = 8
FF = 2048
EPS = 1e-5
ALPHA = 1.0
R = B * D

_LENS = [256, 512, 1024, 2048]


def _dotb(a, b):
    return jnp.dot(a.astype(jnp.bfloat16), b.astype(jnp.bfloat16),
                   preferred_element_type=jnp.float32)


def _gelu(x):
    return jax.nn.gelu(x, approximate=True)


def _bn_rows(x, g_col, be_col, length):
    s1 = jnp.sum(x, axis=1, keepdims=True)
    s2 = jnp.sum(x * x, axis=1, keepdims=True)
    ps1 = s1[:D] + s1[D:]
    ps2 = s2[:D] + s2[D:]
    n = float(B) * length
    m = ps1 / n
    v = ps2 / n - m * m
    scale = jax.lax.rsqrt(v + EPS) * g_col
    shift = be_col - m * scale
    scale2 = jnp.concatenate([scale, scale], axis=0)
    shift2 = jnp.concatenate([shift, shift], axis=0)
    return x * scale2 + shift2


def _pool_mat(n_in, n_out):
    k = n_in // n_out
    ri = jax.lax.broadcasted_iota(jnp.int32, (n_in, n_out), 0)
    ci = jax.lax.broadcasted_iota(jnp.int32, (n_in, n_out), 1)
    return jnp.where(ri // k == ci, 1.0 / k, 0.0).astype(jnp.float32)


def _pool_exact(x, n_out):
    pm = _pool_mat(L, n_out)
    hi = x.astype(jnp.bfloat16).astype(jnp.float32)
    lo = x - hi
    return _dotb(hi, pm) + _dotb(lo, pm)


def _mdm_layer(out, sample, w1, b1, w2, b2, g, be, length):
    h = _gelu(_dotb(out, w1) + b1)
    out = _dotb(h, w2) + b2
    out = out + sample
    return _bn_rows(out, g, be, length)


def _backbone_kernel(x_ref, rw_ref, rb_ref,
                     w10, b10, w20, b20, g0, be0,
                     w11, b11, w21, b21, g1, be1,
                     w12, b12, w22, b22, g2, be2,
                     gw_ref, gb_ref,
                     xn_ref, gates_ref, loss_ref, mean_ref, std_ref):
    X = x_ref[...]
    m = jnp.mean(X, axis=1, keepdims=True)
    ex2 = jnp.mean(X * X, axis=1, keepdims=True)
    std = jnp.sqrt(ex2 - m * m + EPS)
    mean_ref[...] = m
    std_ref[...] = std
    Xn = (X - m) / std * rw_ref[...] + rb_ref[...]
    xn_ref[...] = Xn

    s8 = _pool_exact(Xn, 256)
    s4 = _pool_exact(Xn, 512)
    s2 = _pool_exact(Xn, 1024)
    out = _mdm_layer(s8, s4, w10[...], b10[...], w20[...], b20[...],
                     g0[...], be0[...], float(_LENS[1]))
    out = _mdm_layer(out, s2, w11[...], b11[...], w21[...], b21[...],
                     g1[...], be1[...], float(_LENS[2]))
    te = _mdm_layer(out, Xn, w12[...], b12[...], w22[...], b22[...],
                    g2[...], be2[...], float(_LENS[3]))

    logits = _dotb(te, gw_ref[...]) + gb_ref[...]
    iota = jax.lax.broadcasted_iota(jnp.int32, (R, E), 1)
    m1 = jnp.max(logits, axis=1, keepdims=True)
    i1 = jnp.min(jnp.where(logits == m1, iota, E), axis=1, keepdims=True)
    masked = jnp.where(iota == i1, -jnp.inf, logits)
    m2 = jnp.max(masked, axis=1, keepdims=True)
    i2 = jnp.min(jnp.where(masked == m2, iota, E), axis=1, keepdims=True)
    e2 = jnp.exp(m2 - m1)
    g0v = 1.0 / (1.0 + e2)
    g1v = 1.0 - g0v
    gates = jnp.where(iota == i1, g0v, 0.0) + jnp.where(iota == i2, g1v, 0.0)
    gates_ref[...] = gates
    imp = jnp.sum(gates, axis=0, keepdims=True)
    im = jnp.sum(imp, axis=1, keepdims=True) / E
    iv = jnp.sum(imp * imp, axis=1, keepdims=True) / E - im * im
    loss_ref[...] = iv / (im * im + 1e-10)


def _ddi_kernel(x_ref, w1, b1, w2, b2, wc1t, bc1, wc2t, bc2, g, be, o_ref):
    X = x_ref[...]
    xcat = jnp.concatenate(
        [X[:, j * PATCH:(j + 1) * PATCH] for j in range(NPATCH)], axis=0)
    h = _gelu(_dotb(xcat, w1[...]) + b1[...])
    u = _dotb(h, w2[...]) + b2[...]
    X = X + jnp.concatenate(
        [u[j * R:(j + 1) * R] for j in range(NPATCH)], axis=1)
    outs = []
    for bi in range(B):
        Xb = X[bi * D:(bi + 1) * D]
        Y = _gelu(_dotb(wc1t[...], Xb) + bc1[...])
        V = _dotb(wc2t[...], Y) + bc2[...]
        outs.append(Xb + ALPHA * V)
    X = jnp.concatenate(outs, axis=0)
    o_ref[...] = _bn_rows(X, g[...].T, be[...].T, float(L))


def _moe_kernel(tok_ref, gates_ref, ew1, eb1, ew2, eb2,
                mean_ref, std_ref, rw_ref, rb_ref, out_ref):
    e = pl.program_id(0)
    f = pl.program_id(1)
    nf = pl.num_programs(1)

    @pl.when(jnp.logical_and(e == 0, f == 0))
    def _init():
        out_ref[...] = jnp.zeros_like(out_ref)

    tok = tok_ref[...]
    h = _gelu(_dotb(tok, ew1[0]) + eb1[0])
    part = _dotb(h, ew2[0])
    ei = jax.lax.broadcasted_iota(jnp.int32, (R, E), 1)
    g = jnp.sum(jnp.where(ei == e, gates_ref[...], 0.0), axis=1, keepdims=True)

    @pl.when(f == 0)
    def _bias():
        out_ref[...] += g * eb2[0]

    out_ref[...] += g * part

    @pl.when(jnp.logical_and(e == E - 1, f == nf - 1))
    def _finish():
        o = out_ref[...]
        o = (o - rb_ref[...]) / rw_ref[...] * std_ref[...] + mean_ref[...]
        out_ref[...] = o


def kernel(x, params):
    p = params
    bf = jnp.bfloat16
    xt = jnp.transpose(x, (0, 2, 1)).reshape(R, L)
    rw = jnp.tile(p['revin_w'], B)[:, None]
    rb = jnp.tile(p['revin_b'], B)[:, None]

    vmem = functools.partial(pl.BlockSpec, memory_space=pltpu.VMEM)

    def layer_args(blk):
        return [blk['w1'].astype(bf), blk['b1'][None, :],
                blk['w2'].astype(bf), blk['b2'][None, :],
                blk['g'][:, None], blk['be'][:, None]]

    mdm = p['mdm']
    xn, gates, loss, mean, std = pl.pallas_call(
        _backbone_kernel,
        out_shape=[
            jax.ShapeDtypeStruct((R, L), jnp.float32),
            jax.ShapeDtypeStruct((R, E), jnp.float32),
            jax.ShapeDtypeStruct((1, 1), jnp.float32),
            jax.ShapeDtypeStruct((R, 1), jnp.float32),
            jax.ShapeDtypeStruct((R, 1), jnp.float32),
        ],
        compiler_params=pltpu.CompilerParams(
            vmem_limit_bytes=110 * 1024 * 1024),
    )(xt, rw, rb, *layer_args(mdm[0]), *layer_args(mdm[1]),
      *layer_args(mdm[2]), p['gate_w'].astype(bf), p['gate_b'][None, :])

    cur = xn
    for blk in p['ddi']:
        cur = pl.pallas_call(
            _ddi_kernel,
            out_shape=jax.ShapeDtypeStruct((R, L), jnp.float32),
            compiler_params=pltpu.CompilerParams(
                vmem_limit_bytes=110 * 1024 * 1024),
        )(cur, blk['w1'].astype(bf), blk['b1'][None, :],
          blk['w2'].astype(bf), blk['b2'][None, :],
          blk['wc1'].T.astype(bf), blk['bc1'][:, None],
          blk['wc2'].T.astype(bf), blk['bc2'][:, None],
          blk['g'][None, :], blk['be'][None, :])

    FBLK = 2048
    nf = FF // FBLK
    out_tok = pl.pallas_call(
        _moe_kernel,
        grid=(E, nf),
        in_specs=[
            vmem((R, L), lambda e, f: (0, 0)),
            vmem((R, E), lambda e, f: (0, 0)),
            vmem((1, L, FBLK), lambda e, f: (e, 0, f)),
            vmem((1, 1, FBLK), lambda e, f: (e, 0, f)),
            vmem((1, FBLK, PRED), lambda e, f: (e, f, 0)),
            vmem((1, 1, PRED), lambda e, f: (e, 0, 0)),
            vmem((R, 1), lambda e, f: (0, 0)),
            vmem((R, 1), lambda e, f: (0, 0)),
            vmem((R, 1), lambda e, f: (0, 0)),
            vmem((R, 1), lambda e, f: (0, 0)),
        ],
        out_specs=vmem((R, PRED), lambda e, f: (0, 0)),
        out_shape=jax.ShapeDtypeStruct((R, PRED), jnp.float32),
        compiler_params=pltpu.CompilerParams(
            dimension_semantics=("arbitrary", "arbitrary"),
            vmem_limit_bytes=100 * 1024 * 1024),
    )(cur, gates, p['ew1'].astype(bf), p['eb1'][:, None, :],
      p['ew2'].astype(bf), p['eb2'][:, None, :], mean, std, rw, rb)

    out = jnp.transpose(out_tok.reshape(B, D, PRED), (0, 2, 1))
    return out, loss.reshape(())
